# baseline (speedup 1.0000x reference)
import functools

import jax
import jax.numpy as jnp
from jax import lax
from jax.experimental import pallas as pl
from jax.experimental.pallas import tpu as pltpu
from jax.experimental.pallas import tpu_sc as plsc

K = 9
SUB = 16

_HI = jax.lax.Precision.HIGHEST

_NC = 2
_NS = 16
_LANES = 16


def _fold_kernel(fc2_ref, cw_ref, wf_ref):
    fc2 = fc2_ref[...]
    for k in range(K):
        wf_ref[k] = jnp.dot(fc2, cw_ref[k], precision=_HI,
                            preferred_element_type=jnp.float32)


def _tc_kernel(x_ref, fc1_ref, ss_ref, sb_ref, wf_ref, idx_ref, vt_ref):
    x = x_ref[0]
    fc1 = fc1_ref[...]
    s = ss_ref[0, 0]
    bias = sb_ref[0, 0]
    C2 = fc1.shape[0]
    N = x.shape[1]
    M = N // SUB

    h = jnp.dot(fc1.astype(jnp.bfloat16), x.astype(jnp.bfloat16),
                preferred_element_type=jnp.float32)
    h = s * jnp.square(jnp.maximum(h, 0.0)) + bias

    row = jax.lax.broadcasted_iota(jnp.int32, (N, M), 0)
    col = jax.lax.broadcasted_iota(jnp.int32, (N, M), 1)
    sel_nm = (row == col * SUB).astype(jnp.float32)
    cand = jnp.dot(h, sel_nm, precision=_HI,
                   preferred_element_type=jnp.float32)

    n2 = jnp.sum(h * h, axis=0, keepdims=True)
    csq = cand * cand
    ones = jnp.ones((C2, 1), jnp.float32)
    c2 = jax.lax.dot_general(csq, ones, (((0,), (0,)), ((), ())),
                             precision=_HI,
                             preferred_element_type=jnp.float32)
    d = jax.lax.dot_general(cand.astype(jnp.bfloat16), h.astype(jnp.bfloat16),
                            (((0,), (0,)), ((), ())),
                            preferred_element_type=jnp.float32)
    neg = 2.0 * d - c2 - n2

    iota = jax.lax.broadcasted_iota(jnp.int32, (M, N), 0)
    for k in range(K):
        mx = jnp.max(neg, axis=0, keepdims=True)
        ismax = neg >= mx
        sel = jnp.min(jnp.where(ismax, iota, M), axis=0, keepdims=True)
        idx_ref[0, pl.ds(k, 1), :] = sel + k * M
        onehot = iota == sel
        neg = jnp.where(onehot, -jnp.inf, neg)
        vt_k = jnp.dot(wf_ref[k], cand, precision=_HI,
                       preferred_element_type=jnp.float32)
        vt_ref[0, :, pl.ds(k * M, M)] = vt_k


def _make_sc_gather(B, N, C, M):
    NW = _NC * _NS
    WPB = NW // B
    per_w = N // WPB
    NG = per_w // _LANES
    TBL = K * M
    mesh = plsc.VectorSubcoreMesh(core_axis_name="c", subcore_axis_name="s")

    @functools.partial(
        pl.kernel,
        out_type=jax.ShapeDtypeStruct((B, C, N), jnp.float32),
        mesh=mesh,
        compiler_params=pltpu.CompilerParams(needs_layout_passes=False),
        scratch_types=[
            pltpu.VMEM((K, per_w), jnp.int32),
            pltpu.VMEM((C * TBL,), jnp.float32),
            pltpu.VMEM((C, per_w), jnp.float32),
            pltpu.SemaphoreType.DMA,
            pltpu.SemaphoreType.DMA,
        ],
    )
    def sc_gather(idx_hbm, table_hbm, out_hbm, idx_v, tbl_v, out_v,
                  sem1, sem2):
        wid = lax.axis_index("s") * _NC + lax.axis_index("c")
        b = wid // WPB
        off = (wid % WPB) * per_w
        cp1 = pltpu.async_copy(idx_hbm.at[b, :, pl.ds(off, per_w)], idx_v,
                               sem1)
        cp2 = pltpu.async_copy(table_hbm.at[b], tbl_v, sem2)
        cp1.wait()
        cp2.wait()

        def group_body(g, carry):
            sl = pl.ds(g * _LANES, _LANES)
            ivecs = [idx_v[k, sl] for k in range(K)]
            for j in range(C):
                avecs = [iv + (j * TBL) for iv in ivecs]
                a0 = plsc.load_gather(tbl_v, [avecs[0]])
                a1 = plsc.load_gather(tbl_v, [avecs[1]])
                a2 = plsc.load_gather(tbl_v, [avecs[2]])
                for k in range(3, K):
                    t = plsc.load_gather(tbl_v, [avecs[k]])
                    if k % 3 == 0:
                        a0 = a0 + t
                    elif k % 3 == 1:
                        a1 = a1 + t
                    else:
                        a2 = a2 + t
                out_v[j, sl] = (a0 + a1) + a2
            return carry

        lax.fori_loop(0, NG, group_body, 0)
        pltpu.sync_copy(out_v, out_hbm.at[b, :, pl.ds(off, per_w)])

    return sc_gather


def kernel(x, fc1_w, star_scale, star_bias, conv_w, fc2_w):
    B, C, H, W = x.shape
    N = H * W
    C2 = fc1_w.shape[0]
    M = N // SUB
    xf = x.reshape(B, C, N)
    cw = jnp.transpose(conv_w, (2, 0, 1))

    wf = pl.pallas_call(
        _fold_kernel,
        out_shape=jax.ShapeDtypeStruct((K, C, C2), jnp.float32),
    )(fc2_w, cw)

    ss = jnp.reshape(star_scale, (1, 1)).astype(jnp.float32)
    sb = jnp.reshape(star_bias, (1, 1)).astype(jnp.float32)

    idx, vt = pl.pallas_call(
        _tc_kernel,
        grid=(B,),
        in_specs=[
            pl.BlockSpec((1, C, N), lambda i: (i, 0, 0)),
            pl.BlockSpec((C2, C), lambda i: (0, 0)),
            pl.BlockSpec((1, 1), lambda i: (0, 0)),
            pl.BlockSpec((1, 1), lambda i: (0, 0)),
            pl.BlockSpec((K, C, C2), lambda i: (0, 0, 0)),
        ],
        out_specs=[
            pl.BlockSpec((1, K, N), lambda i: (i, 0, 0)),
            pl.BlockSpec((1, C, K * M), lambda i: (i, 0, 0)),
        ],
        out_shape=[
            jax.ShapeDtypeStruct((B, K, N), jnp.int32),
            jax.ShapeDtypeStruct((B, C, K * M), jnp.float32),
        ],
    )(xf, fc1_w, ss, sb, wf)

    out = _make_sc_gather(B, N, C, M)(idx, vt.reshape(B, C * K * M))
    return out.reshape(B, C, H, W)


# speedup vs baseline: 1.3882x; 1.3882x over previous
import functools

import jax
import jax.numpy as jnp
from jax import lax
from jax.experimental import pallas as pl
from jax.experimental.pallas import tpu as pltpu
from jax.experimental.pallas import tpu_sc as plsc

K = 9
SUB = 16

_HI = jax.lax.Precision.HIGHEST

_NC = 2
_NS = 16
_LANES = 16


def _fold_kernel(fc2_ref, cw_ref, wf_ref):
    fc2 = fc2_ref[...]
    for k in range(K):
        wf_ref[k] = jnp.dot(fc2, cw_ref[k], precision=_HI,
                            preferred_element_type=jnp.float32)


def _tc_kernel(x_ref, fc1_ref, ss_ref, sb_ref, wf_ref, idx_ref, vt_ref):
    x = x_ref[0]
    fc1 = fc1_ref[...]
    s = ss_ref[0, 0]
    bias = sb_ref[0, 0]
    C2 = fc1.shape[0]
    N = x.shape[1]
    M = N // SUB

    h = jnp.dot(fc1.astype(jnp.bfloat16), x.astype(jnp.bfloat16),
                preferred_element_type=jnp.float32)
    h = s * jnp.square(jnp.maximum(h, 0.0)) + bias

    row = jax.lax.broadcasted_iota(jnp.int32, (N, M), 0)
    col = jax.lax.broadcasted_iota(jnp.int32, (N, M), 1)
    sel_nm = (row == col * SUB).astype(jnp.float32)
    cand = jnp.dot(h, sel_nm, precision=_HI,
                   preferred_element_type=jnp.float32)

    n2 = jnp.sum(h * h, axis=0, keepdims=True)
    csq = cand * cand
    ones = jnp.ones((C2, 1), jnp.float32)
    c2 = jax.lax.dot_general(csq, ones, (((0,), (0,)), ((), ())),
                             precision=_HI,
                             preferred_element_type=jnp.float32)
    d = jax.lax.dot_general(cand.astype(jnp.bfloat16), h.astype(jnp.bfloat16),
                            (((0,), (0,)), ((), ())),
                            preferred_element_type=jnp.float32)
    neg = 2.0 * d - c2 - n2

    iota = jax.lax.broadcasted_iota(jnp.int32, (M, N), 0)
    for k in range(K):
        mx = jnp.max(neg, axis=0, keepdims=True)
        ismax = neg >= mx
        sel = jnp.min(jnp.where(ismax, iota, M), axis=0, keepdims=True)
        idx_ref[0, pl.ds(k, 1), :] = sel + k * M
        onehot = iota == sel
        neg = jnp.where(onehot, -jnp.inf, neg)
        vt_k = jnp.dot(wf_ref[k], cand, precision=_HI,
                       preferred_element_type=jnp.float32)
        vt_ref[0, :, pl.ds(k * M, M)] = vt_k


def _make_sc_gather(B, N, C, M):
    NW = _NC * _NS
    WPB = NW // B
    per_w = N // WPB
    NG = per_w // _LANES
    TBL = K * M
    mesh = plsc.VectorSubcoreMesh(core_axis_name="c", subcore_axis_name="s")

    @functools.partial(
        pl.kernel,
        out_type=jax.ShapeDtypeStruct((B, C, N), jnp.float32),
        mesh=mesh,
        compiler_params=pltpu.CompilerParams(needs_layout_passes=False),
        scratch_types=[
            pltpu.VMEM((K, per_w), jnp.int32),
            pltpu.VMEM(((C // 2) * TBL,), jnp.int32),
            pltpu.VMEM((C, per_w), jnp.float32),
            pltpu.SemaphoreType.DMA,
            pltpu.SemaphoreType.DMA,
        ],
    )
    def sc_gather(idx_hbm, table_hbm, out_hbm, idx_v, tbl_v, out_v,
                  sem1, sem2):
        wid = lax.axis_index("s") * _NC + lax.axis_index("c")
        b = wid // WPB
        off = (wid % WPB) * per_w
        cp1 = pltpu.async_copy(idx_hbm.at[b, :, pl.ds(off, per_w)], idx_v,
                               sem1)
        cp2 = pltpu.async_copy(table_hbm.at[b], tbl_v, sem2)
        cp1.wait()
        cp2.wait()

        def group_body(g, carry):
            sl = pl.ds(g * _LANES, _LANES)
            ivecs = [idx_v[k, sl] for k in range(K)]
            for j2 in range(C // 2):
                evens = []
                odds = []
                for k in range(K):
                    w = plsc.load_gather(tbl_v, [ivecs[k] + (j2 * TBL)])
                    bc = plsc.bitcast(w, jnp.bfloat16)
                    a, bvec = plsc.unpack(
                        bc, format=plsc.PackFormat.INTERLEAVED,
                        preferred_element_type=jnp.float32)
                    evens.append(a)
                    odds.append(bvec)
                for vals, j in ((evens, 2 * j2), (odds, 2 * j2 + 1)):
                    s0 = (vals[0] + vals[3]) + vals[6]
                    s1 = (vals[1] + vals[4]) + vals[7]
                    s2 = (vals[2] + vals[5]) + vals[8]
                    out_v[j, sl] = (s0 + s1) + s2
            return carry

        lax.fori_loop(0, NG, group_body, 0)
        pltpu.sync_copy(out_v, out_hbm.at[b, :, pl.ds(off, per_w)])

    return sc_gather


def kernel(x, fc1_w, star_scale, star_bias, conv_w, fc2_w):
    B, C, H, W = x.shape
    N = H * W
    C2 = fc1_w.shape[0]
    M = N // SUB
    xf = x.reshape(B, C, N)
    cw = jnp.transpose(conv_w, (2, 0, 1))

    wf = pl.pallas_call(
        _fold_kernel,
        out_shape=jax.ShapeDtypeStruct((K, C, C2), jnp.float32),
    )(fc2_w, cw)

    ss = jnp.reshape(star_scale, (1, 1)).astype(jnp.float32)
    sb = jnp.reshape(star_bias, (1, 1)).astype(jnp.float32)

    idx, vt = pl.pallas_call(
        _tc_kernel,
        grid=(B,),
        in_specs=[
            pl.BlockSpec((1, C, N), lambda i: (i, 0, 0)),
            pl.BlockSpec((C2, C), lambda i: (0, 0)),
            pl.BlockSpec((1, 1), lambda i: (0, 0)),
            pl.BlockSpec((1, 1), lambda i: (0, 0)),
            pl.BlockSpec((K, C, C2), lambda i: (0, 0, 0)),
        ],
        out_specs=[
            pl.BlockSpec((1, K, N), lambda i: (i, 0, 0)),
            pl.BlockSpec((1, C, K * M), lambda i: (i, 0, 0)),
        ],
        out_shape=[
            jax.ShapeDtypeStruct((B, K, N), jnp.int32),
            jax.ShapeDtypeStruct((B, C, K * M), jnp.float32),
        ],
    )(xf, fc1_w, ss, sb, wf)

    vtb = vt.astype(jnp.bfloat16)
    packed = jnp.stack([vtb[:, 0::2, :], vtb[:, 1::2, :]], axis=-1)
    tbl_i32 = jax.lax.bitcast_convert_type(packed, jnp.int32)

    out = _make_sc_gather(B, N, C, M)(idx, tbl_i32.reshape(B, (C // 2) * K * M))
    return out.reshape(B, C, H, W)
